# Initial kernel scaffold; baseline (speedup 1.0000x reference)
import jax
import jax.numpy as jnp
from jax.experimental import pallas as pl


def kernel(Xi, Xv, W1, b1, E1, W2, b2, E2, L1_w, L1_b, g1, be1, L2_w, L2_b, g2, be2, bias):
    raise NotImplementedError("write your pallas kernel here")


# speedup vs baseline: 1.3411x; 1.3411x over previous
import jax
import jax.numpy as jnp
from jax import lax
from jax.experimental import pallas as pl
from jax.experimental.pallas import tpu as pltpu
from jax.experimental.pallas import tpu_sc as plsc

B = 1024
FD = 13
FS = 26
V = 1000
D = 128
H0 = (FD + FS) * D
H1 = 2048
H2 = 1024
NROWS = B * FS
NTAB = FS * V
_EPS = 1e-5

_NC = 2
_NS = 16
_NW = _NC * _NS
_RPW = NROWS // _NW
_CH = 64
_NCH = _RPW // _CH


def _rowsum_body(e1_ref, r1_ref):
    r1_ref[...] = jnp.sum(e1_ref[...], axis=2)


def _rowsum(e1):
    return pl.pallas_call(
        _rowsum_body,
        out_shape=jax.ShapeDtypeStruct((FS, V), jnp.float32),
    )(e1)


def _sc_body(e2_ref, r1_ref, idx_ref, g_ref, r1g_ref,
             idx_v, rows_v, r1g_v, sem_r, sem_s):
    c = lax.axis_index("c")
    s = lax.axis_index("s")
    wid = s * _NC + c
    base = wid * _RPW
    pltpu.sync_copy(idx_ref.at[pl.ds(base, _RPW)], idx_v)
    row_cps = []
    sca_cps = []
    for ch in range(_NCH):
        iv = idx_v.at[pl.ds(ch * _CH, _CH)]
        row_cps.append(pltpu.async_copy(
            e2_ref.at[iv], rows_v.at[pl.ds(ch * _CH, _CH)], sem_r))
        sca_cps.append(pltpu.async_copy(
            r1_ref.at[iv], r1g_v.at[pl.ds(ch * _CH, _CH)], sem_s))
    for cp in row_cps:
        cp.wait()
    for cp in sca_cps:
        cp.wait()
    pltpu.sync_copy(rows_v, g_ref.at[pl.ds(base, _RPW)])
    pltpu.sync_copy(r1g_v, r1g_ref.at[pl.ds(base, _RPW)])


def _sc_gather(e2_flat, r1_flat, idx):
    mesh = plsc.VectorSubcoreMesh(core_axis_name="c", subcore_axis_name="s")
    fn = pl.kernel(
        _sc_body,
        out_type=[
            jax.ShapeDtypeStruct((NROWS, D), jnp.float32),
            jax.ShapeDtypeStruct((NROWS,), jnp.float32),
        ],
        mesh=mesh,
        scratch_types=[
            pltpu.VMEM((_RPW,), jnp.int32),
            pltpu.VMEM((_RPW, D), jnp.float32),
            pltpu.VMEM((_RPW,), jnp.float32),
            pltpu.SemaphoreType.DMA,
            pltpu.SemaphoreType.DMA,
        ],
    )
    return fn(e2_flat, r1_flat, idx)


def _mlp1_body(xi_ref, xvd_ref, xvs_ref, w1_ref, b1_ref, w2_ref, b2_ref,
               r1g_ref, sp_ref, l1w_ref, l1b_ref, g1_ref, be1_ref,
               h1_ref, fm_ref, deep_scr):
    jb = pl.program_id(0)

    @pl.when(jb == 0)
    def _prep():
        xi = xi_ref[...]
        xvd = xvd_ref[...]
        xvs = xvs_ref[...]
        a = xi * xvd
        s = jnp.zeros((B, D), jnp.float32)
        ssq = jnp.zeros((B, D), jnp.float32)
        for f in range(FD):
            slab = (a[:, f:f + 1] * w2_ref[f:f + 1, :]
                    + xvd[:, f:f + 1] * b2_ref[f:f + 1, :])
            deep_scr[:, f * D:(f + 1) * D] = slab.astype(jnp.bfloat16)
            s = s + slab
            ssq = ssq + slab * slab
        for f in range(FS):
            g = sp_ref[:, f * D:(f + 1) * D] * xvs[:, f:f + 1]
            deep_scr[:, (FD + f) * D:(FD + f + 1) * D] = g.astype(jnp.bfloat16)
            s = s + g
            ssq = ssq + g * g
        fm2 = 0.5 * jnp.sum(s * s - ssq, axis=1, keepdims=True)
        w1s = jnp.sum(w1_ref[...], axis=1, keepdims=True)
        b1s = jnp.sum(b1_ref[...], axis=1, keepdims=True)
        fm1d = (jnp.dot(a, w1s, preferred_element_type=jnp.float32)
                + jnp.dot(xvd, b1s, preferred_element_type=jnp.float32))
        fm1s = jnp.sum(xvs * r1g_ref[...], axis=1, keepdims=True)
        fm_ref[...] = fm1d + fm1s + fm2

    x1 = jnp.dot(deep_scr[...], l1w_ref[...],
                 preferred_element_type=jnp.float32)
    x1 = x1 + l1b_ref[...][None, :]
    m = jnp.mean(x1, axis=0, keepdims=True)
    v = jnp.mean(x1 * x1, axis=0, keepdims=True) - m * m
    h = (x1 - m) * (g1_ref[...][None, :] * lax.rsqrt(v + _EPS))
    h = h + be1_ref[...][None, :]
    h1_ref[...] = h.astype(jnp.bfloat16)


def _mlp1(xi_f, xvd, xvs, w1, b1, w2, b2, r1g, sp, l1w_bf, l1b, g1, be1):
    nj = 4
    jblk = H1 // nj
    full2 = lambda shape: pl.BlockSpec(shape, lambda j: (0, 0))
    return pl.pallas_call(
        _mlp1_body,
        grid=(nj,),
        in_specs=[
            full2((B, FD)),
            full2((B, FD)),
            full2((B, FS)),
            full2((FD, D)),
            full2((FD, D)),
            full2((FD, D)),
            full2((FD, D)),
            full2((B, FS)),
            full2((B, FS * D)),
            pl.BlockSpec((H0, jblk), lambda j: (0, j)),
            pl.BlockSpec((jblk,), lambda j: (j,)),
            pl.BlockSpec((jblk,), lambda j: (j,)),
            pl.BlockSpec((jblk,), lambda j: (j,)),
        ],
        out_specs=[
            pl.BlockSpec((B, jblk), lambda j: (0, j)),
            pl.BlockSpec((B, 1), lambda j: (0, 0)),
        ],
        out_shape=[
            jax.ShapeDtypeStruct((B, H1), jnp.bfloat16),
            jax.ShapeDtypeStruct((B, 1), jnp.float32),
        ],
        scratch_shapes=[pltpu.VMEM((B, H0), jnp.bfloat16)],
        compiler_params=pltpu.CompilerParams(
            dimension_semantics=("arbitrary",)),
    )(xi_f, xvd, xvs, w1, b1, w2, b2, r1g, sp, l1w_bf, l1b, g1, be1)


def _mlp2_body(h1_ref, l2w_ref, l2b_ref, g2_ref, be2_ref, fm_ref, bias_ref,
               out_ref):
    x2 = jnp.dot(h1_ref[...], l2w_ref[...],
                 preferred_element_type=jnp.float32)
    x2 = x2 + l2b_ref[...][None, :]
    m = jnp.mean(x2, axis=0, keepdims=True)
    v = jnp.mean(x2 * x2, axis=0, keepdims=True) - m * m
    coef = g2_ref[...][None, :] * lax.rsqrt(v + _EPS)
    hsum = (jnp.sum((x2 - m) * coef, axis=1, keepdims=True)
            + jnp.sum(be2_ref[...]))
    out_ref[...] = hsum + fm_ref[...] + bias_ref[...]


def _mlp2(h1, l2w_bf, l2b, g2, be2, fm, bias_col):
    return pl.pallas_call(
        _mlp2_body,
        out_shape=jax.ShapeDtypeStruct((B, 1), jnp.float32),
    )(h1, l2w_bf, l2b, g2, be2, fm, bias_col)


def kernel(Xi, Xv, W1, b1, E1, W2, b2, E2, L1_w, L1_b, g1, be1,
           L2_w, L2_b, g2, be2, bias):
    xi_f = Xi[:, :FD, 0].astype(jnp.float32)
    xi_s = Xi[:, FD:, 0].astype(jnp.int32)
    xvd = Xv[:, :FD]
    xvs = Xv[:, FD:]
    idx = (xi_s + (jnp.arange(FS, dtype=jnp.int32) * V)[None, :]).reshape(NROWS)
    r1 = _rowsum(E1).reshape(NTAB)
    grows, r1g_flat = _sc_gather(E2.reshape(NTAB, D), r1, idx)
    sp = grows.reshape(B, FS * D)
    r1g = r1g_flat.reshape(B, FS)
    h1, fm = _mlp1(xi_f, xvd, xvs, W1[:, 0, :], b1, W2[:, 0, :], b2,
                   r1g, sp, L1_w.astype(jnp.bfloat16), L1_b, g1, be1)
    out = _mlp2(h1, L2_w.astype(jnp.bfloat16), L2_b, g2, be2, fm,
                bias.reshape(B, 1))
    return out.reshape(B)
